# Initial kernel scaffold; baseline (speedup 1.0000x reference)
import jax
import jax.numpy as jnp
from jax.experimental import pallas as pl


def kernel(x, W, b):
    raise NotImplementedError("write your pallas kernel here")


# speedup vs baseline: 2.6170x; 2.6170x over previous
import functools

import jax
import jax.numpy as jnp
from jax.experimental import pallas as pl


def _routing_body(x_ref, w_ref, b_ref, gates_ref, idx_ref, val_ref):
    xb = x_ref[0]
    w = w_ref[...]
    logits = jnp.dot(w, xb, preferred_element_type=jnp.float32)
    logits = logits + b_ref[...]

    E = logits.shape[0]
    eiota = jax.lax.broadcasted_iota(jnp.int32, logits.shape, 0)

    m1 = jnp.max(logits, axis=0, keepdims=True)
    i1 = jnp.min(jnp.where(logits == m1, eiota, E), axis=0, keepdims=True)
    masked = jnp.where(eiota == i1, -jnp.inf, logits)
    m2 = jnp.max(masked, axis=0, keepdims=True)
    i2 = jnp.min(jnp.where(masked == m2, eiota, E), axis=0, keepdims=True)

    ex = jnp.exp(logits - m1)
    recip = 1.0 / jnp.sum(ex, axis=0, keepdims=True)
    v1 = recip
    v2 = jnp.exp(m2 - m1) * recip

    zero = jnp.zeros_like(logits)
    gates_ref[0] = (jnp.where(eiota == i1, v1, zero)
                    + jnp.where(eiota == i2, v2, zero))
    idx_ref[0] = jnp.concatenate([i1, i2], axis=0)
    val_ref[0] = jnp.concatenate([v1, v2], axis=0)


def _pick_tile(s):
    for t in (3584, 1792, 1024, 512, 256, 128):
        if s % t == 0:
            return t
    return s


@functools.partial(jax.jit, static_argnames=())
def kernel(x, W, b):
    B, C, H, Wd = x.shape
    E = W.shape[0]
    S = H * Wd
    xr = x.reshape(B, C, S)
    b2 = b.reshape(E, 1)
    T = _pick_tile(S)
    grid = (B, S // T)

    gates, idx, vals = pl.pallas_call(
        _routing_body,
        grid=grid,
        in_specs=[
            pl.BlockSpec((1, C, T), lambda bi, ti: (bi, 0, ti)),
            pl.BlockSpec((E, C), lambda bi, ti: (0, 0)),
            pl.BlockSpec((E, 1), lambda bi, ti: (0, 0)),
        ],
        out_specs=[
            pl.BlockSpec((1, E, T), lambda bi, ti: (bi, 0, ti)),
            pl.BlockSpec((1, 2, T), lambda bi, ti: (bi, 0, ti)),
            pl.BlockSpec((1, 2, T), lambda bi, ti: (bi, 0, ti)),
        ],
        out_shape=[
            jax.ShapeDtypeStruct((B, E, S), jnp.float32),
            jax.ShapeDtypeStruct((B, 2, S), jnp.int32),
            jax.ShapeDtypeStruct((B, 2, S), jnp.float32),
        ],
    )(xr, W, b2)

    return (gates.reshape(B, E, H, Wd),
            idx.reshape(B, 2, H, Wd),
            vals.reshape(B, 2, H, Wd))
